# baseline (speedup 1.0000x reference)
import functools

import jax
import jax.numpy as jnp
from jax.experimental import pallas as pl
import jax.experimental.pallas.tpu as pltpu

_N = 4096
_D = 64
_ROWS = 512
_NCHUNK = _N // _ROWS
_NBUF = 3


def _body(x_ref, y_ref, b_ref, vkyw_ref, vkyb_ref, wkxw_ref, wkxb_ref,
          vkxw_ref, vkxb_ref, sig_ref, a_hbm, out_ref, abuf, sems):

    def copy(i):
        return pltpu.make_async_copy(
            a_hbm.at[pl.ds(i * _ROWS, _ROWS), :],
            abuf.at[i % _NBUF],
            sems.at[i % _NBUF],
        )

    for j in range(_NBUF):
        copy(j).start()

    cw = vkxw_ref[...] - 2.0 * wkxw_ref[...]
    cb = vkxb_ref[...] - 2.0 * wkxb_ref[...]
    u = jnp.dot(x_ref[...], cw.T, preferred_element_type=jnp.float32) + cb
    u_bf = u.astype(jnp.bfloat16)
    sig = sig_ref[0, 0]
    vky = (
        jnp.dot(y_ref[...], vkyw_ref[...].T, preferred_element_type=jnp.float32)
        + vkyb_ref[...]
    )
    r = vky - sig * b_ref[...]

    for i in range(_NCHUNK):
        copy(i).wait()
        t = jnp.dot(
            abuf[i % _NBUF].astype(jnp.bfloat16),
            u_bf,
            preferred_element_type=jnp.float32,
        )
        if i + _NBUF < _NCHUNK:
            copy(i + _NBUF).start()
        out_ref[pl.ds(i * _ROWS, _ROWS), :] = jnp.maximum(
            r[i * _ROWS:(i + 1) * _ROWS, :] - sig * t, 0.0
        )


@functools.partial(jax.jit, static_argnames=())
def kernel(x, y, A, b, Vky_W, Vky_b, Wkx_W, Wkx_b, Vkx_W, Vkx_b, sigma):
    n, d = x.shape

    vmem = lambda: pl.BlockSpec(memory_space=pltpu.VMEM)

    out = pl.pallas_call(
        _body,
        in_specs=[
            vmem(),
            vmem(),
            vmem(),
            vmem(),
            vmem(),
            vmem(),
            vmem(),
            vmem(),
            vmem(),
            pl.BlockSpec(memory_space=pltpu.SMEM),
            pl.BlockSpec(memory_space=pl.ANY),
        ],
        out_specs=vmem(),
        out_shape=jax.ShapeDtypeStruct((n, d), jnp.float32),
        scratch_shapes=[
            pltpu.VMEM((_NBUF, _ROWS, n), jnp.float32),
            pltpu.SemaphoreType.DMA((_NBUF,)),
        ],
    )(
        x, y, b,
        Vky_W, Vky_b.reshape(1, d),
        Wkx_W, Wkx_b.reshape(1, d),
        Vkx_W, Vkx_b.reshape(1, d),
        sigma.reshape(1, 1),
        A,
    )
    return out


# speedup vs baseline: 1.0901x; 1.0901x over previous
import functools

import jax
import jax.numpy as jnp
from jax.experimental import pallas as pl
import jax.experimental.pallas.tpu as pltpu


def _body(x_ref, y_ref, a_ref, b_ref, vkyw_ref, vkyb_ref, wkxw_ref,
          wkxb_ref, vkxw_ref, vkxb_ref, sig_ref, out_ref, u_ref):
    i = pl.program_id(0)

    @pl.when(i == 0)
    def _compute_u():
        cw = vkxw_ref[...] - 2.0 * wkxw_ref[...]
        cb = vkxb_ref[...] - 2.0 * wkxb_ref[...]
        u_ref[...] = (
            jnp.dot(x_ref[...], cw.T, preferred_element_type=jnp.float32) + cb
        ).astype(jnp.bfloat16)

    t = b_ref[...] + jnp.dot(
        a_ref[...].astype(jnp.bfloat16),
        u_ref[...],
        preferred_element_type=jnp.float32,
    )
    vky = (
        jnp.dot(y_ref[...], vkyw_ref[...].T, preferred_element_type=jnp.float32)
        + vkyb_ref[...]
    )
    out_ref[...] = jnp.maximum(vky - sig_ref[0, 0] * t, 0.0)


@functools.partial(jax.jit, static_argnames=())
def kernel(x, y, A, b, Vky_W, Vky_b, Wkx_W, Wkx_b, Vkx_W, Vkx_b, sigma):
    n, d = x.shape
    bm = 512
    grid = (n // bm,)

    full = lambda shape: pl.BlockSpec(shape, lambda i: (0, 0))
    row_blk = lambda w: pl.BlockSpec((bm, w), lambda i: (i, 0))

    out = pl.pallas_call(
        _body,
        grid=grid,
        in_specs=[
            full((n, d)),
            row_blk(d),
            row_blk(n),
            row_blk(1),
            full((d, d)),
            full((1, d)),
            full((d, d)),
            full((1, d)),
            full((d, d)),
            full((1, d)),
            pl.BlockSpec(memory_space=pltpu.SMEM),
        ],
        out_specs=row_blk(d),
        out_shape=jax.ShapeDtypeStruct((n, d), jnp.float32),
        scratch_shapes=[pltpu.VMEM((n, d), jnp.bfloat16)],
    )(
        x, y, A, b,
        Vky_W, Vky_b.reshape(1, d),
        Wkx_W, Wkx_b.reshape(1, d),
        Vkx_W, Vkx_b.reshape(1, d),
        sigma.reshape(1, 1),
    )
    return out


# speedup vs baseline: 1.1041x; 1.0128x over previous
import functools

import jax
import jax.numpy as jnp
from jax.experimental import pallas as pl
import jax.experimental.pallas.tpu as pltpu


def _body(x_ref, y_ref, a_ref, b_ref, vkyw_ref, vkyb_ref, wkxw_ref,
          wkxb_ref, vkxw_ref, vkxb_ref, sig_ref, out_ref, u_ref):
    i = pl.program_id(0)

    @pl.when(i == 0)
    def _compute_u():
        cw = vkxw_ref[...] - 2.0 * wkxw_ref[...]
        cb = vkxb_ref[...] - 2.0 * wkxb_ref[...]
        u_ref[...] = (
            jnp.dot(x_ref[...], cw.T, preferred_element_type=jnp.float32)
            + cb[None, :]
        ).astype(jnp.bfloat16)

    t = b_ref[...] + jnp.dot(
        a_ref[...].astype(jnp.bfloat16),
        u_ref[...],
        preferred_element_type=jnp.float32,
    )
    vky = (
        jnp.dot(y_ref[...], vkyw_ref[...].T, preferred_element_type=jnp.float32)
        + vkyb_ref[...][None, :]
    )
    out_ref[...] = jnp.maximum(vky - sig_ref[0] * t, 0.0)


@functools.partial(jax.jit, static_argnames=())
def kernel(x, y, A, b, Vky_W, Vky_b, Wkx_W, Wkx_b, Vkx_W, Vkx_b, sigma):
    n, d = x.shape
    bm = 512
    grid = (n // bm,)

    full = lambda shape: pl.BlockSpec(shape, lambda i: (0, 0))
    row_blk = lambda w: pl.BlockSpec((bm, w), lambda i: (i, 0))

    out = pl.pallas_call(
        _body,
        grid=grid,
        in_specs=[
            full((n, d)),
            row_blk(d),
            row_blk(n),
            row_blk(1),
            full((d, d)),
            pl.BlockSpec(memory_space=pltpu.VMEM),
            full((d, d)),
            pl.BlockSpec(memory_space=pltpu.VMEM),
            full((d, d)),
            pl.BlockSpec(memory_space=pltpu.VMEM),
            pl.BlockSpec(memory_space=pltpu.SMEM),
        ],
        out_specs=row_blk(d),
        out_shape=jax.ShapeDtypeStruct((n, d), jnp.float32),
        scratch_shapes=[pltpu.VMEM((n, d), jnp.bfloat16)],
    )(
        x, y, A, b,
        Vky_W, Vky_b,
        Wkx_W, Wkx_b,
        Vkx_W, Vkx_b,
        sigma,
    )
    return out


# speedup vs baseline: 1.5774x; 1.4286x over previous
import functools

import jax
import jax.numpy as jnp
from jax.experimental import pallas as pl
import jax.experimental.pallas.tpu as pltpu


def _body(xt_ref, yt_ref, a_ref, b_ref, vkyw_ref, vkyb_ref, wkxw_ref,
          wkxb_ref, vkxw_ref, vkxb_ref, sig_ref, out_ref, u_ref, vky_ref):
    i = pl.program_id(0)
    bm = a_ref.shape[0]

    @pl.when(i == 0)
    def _prologue():
        cw = vkxw_ref[...] - 2.0 * wkxw_ref[...]
        cb = vkxb_ref[...] - 2.0 * wkxb_ref[...]
        u_ref[...] = (
            jax.lax.dot_general(
                xt_ref[...], cw,
                (((0,), (1,)), ((), ())),
                preferred_element_type=jnp.float32,
            )
            + cb[None, :]
        ).astype(jnp.bfloat16)
        vky_ref[...] = (
            jnp.dot(vkyw_ref[...], yt_ref[...],
                    preferred_element_type=jnp.float32)
            + vkyb_ref[...][:, None]
        )

    t = jnp.dot(
        a_ref[...].astype(jnp.bfloat16),
        u_ref[...],
        preferred_element_type=jnp.float32,
    )
    tt = t.T
    vky_sl = vky_ref[:, pl.ds(i * bm, bm)]
    b_sl = b_ref[pl.ds(i * bm, bm)]
    out_ref[...] = jnp.maximum(
        vky_sl - sig_ref[0] * (b_sl[None, :] + tt), 0.0
    )


@functools.partial(jax.jit, static_argnames=())
def kernel(x, y, A, b, Vky_W, Vky_b, Wkx_W, Wkx_b, Vkx_W, Vkx_b, sigma):
    n, d = x.shape
    bm = 512
    grid = (n // bm,)

    full = lambda shape: pl.BlockSpec(shape, lambda i: (0, 0))
    anyb = lambda: pl.BlockSpec(memory_space=pltpu.VMEM)

    out_t = pl.pallas_call(
        _body,
        grid=grid,
        in_specs=[
            full((d, n)),
            full((d, n)),
            pl.BlockSpec((bm, n), lambda i: (i, 0)),
            anyb(),
            full((d, d)),
            anyb(),
            full((d, d)),
            anyb(),
            full((d, d)),
            anyb(),
            pl.BlockSpec(memory_space=pltpu.SMEM),
        ],
        out_specs=pl.BlockSpec((d, bm), lambda i: (0, i)),
        out_shape=jax.ShapeDtypeStruct((d, n), jnp.float32),
        scratch_shapes=[
            pltpu.VMEM((n, d), jnp.bfloat16),
            pltpu.VMEM((d, n), jnp.float32),
        ],
    )(
        x.T, y.T, A, b.reshape(n),
        Vky_W, Vky_b,
        Wkx_W, Wkx_b,
        Vkx_W, Vkx_b,
        sigma,
    )
    return out_t.T
